# baseline (speedup 1.0000x reference)
import functools

import jax
import jax.numpy as jnp
from jax import lax
from jax.experimental import pallas as pl
from jax.experimental.pallas import tpu as pltpu
from jax.experimental.pallas import tpu_sc as plsc

N_NODES = 10000
N_EDGES = 320000
D = 128

NC, NS = 2, 16
NW = NC * NS
EPT = N_EDGES // NW
CHUNK = 128
NCHUNKS = 80
HALF = NCHUNKS // 2
EPT_P = NCHUNKS * CHUNK
N_PAD = 10240
ROWS_PT = N_PAD // NS


def _sc_aggregate(x, src_f, dst_f, zf, zc, ones):
  mesh = plsc.VectorSubcoreMesh(
      core_axis_name="c", subcore_axis_name="s", num_cores=NC, num_subcores=NS)

  @functools.partial(
      pl.kernel,
      out_type=(
          jax.ShapeDtypeStruct((NC, N_PAD, D), jnp.float32),
          jax.ShapeDtypeStruct((NC, N_PAD), jnp.float32),
      ),
      mesh=mesh,
      scratch_types=[
          pltpu.VMEM((HALF * CHUNK,), jnp.int32),
          pltpu.VMEM((NCHUNKS, CHUNK), jnp.int32),
          pltpu.VMEM((2, CHUNK, D), jnp.float32),
          pltpu.VMEM((CHUNK,), jnp.float32),
          pltpu.VMEM_SHARED((N_PAD, D), jnp.float32),
          pltpu.VMEM_SHARED((N_PAD,), jnp.float32),
          pltpu.SemaphoreType.DMA,
          pltpu.SemaphoreType.DMA,
          pltpu.SemaphoreType.DMA,
      ],
  )
  def agg(x_hbm, src_hbm, dst_hbm, zf_hbm, zc_hbm, ones_hbm, of_hbm, oc_hbm,
          src_v, dst_v, rows_v, ones_v, acc_f, acc_c, sem0, sem1, semi):
    c = lax.axis_index("c")
    s = lax.axis_index("s")
    wid = c * NS + s
    row0 = s * ROWS_PT
    ebase = wid * EPT_P

    def fire(j, carry):
      pltpu.async_copy(dst_hbm.at[pl.ds(ebase + j * CHUNK, CHUNK)],
                       dst_v.at[j], semi)
      return carry
    lax.fori_loop(0, NCHUNKS, fire, 0)
    pltpu.sync_copy(src_hbm.at[pl.ds(ebase, HALF * CHUNK)], src_v)
    pltpu.sync_copy(ones_hbm, ones_v)

    def zero_rows(i, carry):
      r = row0 + i * 128
      pltpu.sync_copy(zf_hbm.at[pl.ds(r, 128)], acc_f.at[pl.ds(r, 128)])
      return carry
    lax.fori_loop(0, ROWS_PT // 128, zero_rows, 0)
    pltpu.sync_copy(zc_hbm.at[pl.ds(row0, ROWS_PT)],
                    acc_c.at[pl.ds(row0, ROWS_PT)])

    def drain(j, carry):
      pltpu.make_async_copy(dst_hbm.at[pl.ds(ebase + j * CHUNK, CHUNK)],
                            dst_v.at[j], semi).wait()
      return carry
    lax.fori_loop(0, NCHUNKS, drain, 0)

    pltpu.async_copy(x_hbm.at[src_v.at[pl.ds(0, CHUNK)]], rows_v.at[0], sem0)

    plsc.subcore_barrier()

    def step(j, carry):
      jm = lax.rem(j, 2)
      jj = lax.rem(j, HALF)
      pltpu.make_async_copy(
          x_hbm.at[src_v.at[pl.ds(jj * CHUNK, CHUNK)]], rows_v.at[jm],
          sem0).wait()

      @pl.when(j == HALF - 1)
      def _():
        pltpu.sync_copy(src_hbm.at[pl.ds(ebase + HALF * CHUNK, HALF * CHUNK)],
                        src_v)

      @pl.when(j > 0)
      def _():
        jp = j - 1
        jpm = lax.rem(jp, 2)
        pltpu.make_async_copy(rows_v.at[jpm], acc_f.at[dst_v.at[jp]],
                              sem1).wait()
        pltpu.make_async_copy(ones_v, acc_c.at[dst_v.at[jp]], sem1).wait()

      pltpu.async_copy(rows_v.at[jm], acc_f.at[dst_v.at[j]], sem1, add=True)
      pltpu.async_copy(ones_v, acc_c.at[dst_v.at[j]], sem1, add=True)

      @pl.when(j + 1 < NCHUNKS)
      def _():
        jn = lax.rem(j + 1, HALF)
        pltpu.async_copy(x_hbm.at[src_v.at[pl.ds(jn * CHUNK, CHUNK)]],
                         rows_v.at[lax.rem(j + 1, 2)], sem0)
      return carry
    lax.fori_loop(0, NCHUNKS, step, 0)

    jl = NCHUNKS - 1
    pltpu.make_async_copy(rows_v.at[lax.rem(jl, 2)], acc_f.at[dst_v.at[jl]],
                          sem1).wait()
    pltpu.make_async_copy(ones_v, acc_c.at[dst_v.at[jl]], sem1).wait()

    plsc.subcore_barrier()
    def write_rows(i, carry):
      r = row0 + i * 128
      pltpu.sync_copy(acc_f.at[pl.ds(r, 128)], of_hbm.at[c, pl.ds(r, 128)])
      return carry
    lax.fori_loop(0, ROWS_PT // 128, write_rows, 0)
    pltpu.sync_copy(acc_c.at[pl.ds(row0, ROWS_PT)],
                    oc_hbm.at[c, pl.ds(row0, ROWS_PT)])

  return agg(x, src_f, dst_f, zf, zc, ones)


def _tc_finish(x, pf, pc, W, b2):
  blk = 1000

  def body(pf_ref, pc_ref, x_ref, w_ref, b_ref, o_ref):
    ssum = pf_ref[0] + pf_ref[1]
    cnt = pc_ref[:, 0:1] + pc_ref[:, 1:2]
    mean = ssum / jnp.maximum(cnt, 1.0)
    h = jnp.where(cnt > 0.0, mean, x_ref[...])
    y = lax.dot_general(h, w_ref[...], (((1,), (1,)), ((), ())),
                        preferred_element_type=jnp.float32)
    o_ref[...] = jnp.maximum(y + b_ref[...], 0.0)

  return pl.pallas_call(
      body,
      grid=(N_NODES // blk,),
      in_specs=[
          pl.BlockSpec((2, blk, D), lambda i: (0, i, 0)),
          pl.BlockSpec((blk, 2), lambda i: (i, 0)),
          pl.BlockSpec((blk, D), lambda i: (i, 0)),
          pl.BlockSpec((D, D), lambda i: (0, 0)),
          pl.BlockSpec((1, D), lambda i: (0, 0)),
      ],
      out_specs=pl.BlockSpec((blk, D), lambda i: (i, 0)),
      out_shape=jax.ShapeDtypeStruct((N_NODES, D), jnp.float32),
  )(pf, pc, x, W, b2)


def kernel(x, edge_index, W, b):
  ei = edge_index.astype(jnp.int32)
  pad = ((0, 0), (0, EPT_P - EPT))
  src_f = jnp.pad(ei[0].reshape(NW, EPT), pad,
                  constant_values=0).reshape(NW * EPT_P)
  dst_f = jnp.pad(ei[1].reshape(NW, EPT), pad,
                  constant_values=N_NODES).reshape(NW * EPT_P)
  zf = jnp.zeros((N_PAD, D), jnp.float32)
  zc = jnp.zeros((N_PAD,), jnp.float32)
  ones = jnp.ones((CHUNK,), jnp.float32)
  pf, pc = _sc_aggregate(x, src_f, dst_f, zf, zc, ones)
  return _tc_finish(x, pf, pc.T, W, b.reshape(1, D))


# speedup vs baseline: 1.9848x; 1.9848x over previous
import functools

import jax
import jax.numpy as jnp
from jax import lax
from jax.experimental import pallas as pl
from jax.experimental.pallas import tpu as pltpu
from jax.experimental.pallas import tpu_sc as plsc

N_NODES = 10000
N_EDGES = 320000
D = 128

NC, NS = 2, 16
NW = NC * NS
EPT = N_EDGES // NW
CHUNK = 88
NCHUNKS = 114
EPT_P = NCHUNKS * CHUNK
N_PAD = 10240
ROWS_PT = N_PAD // NS


def _sc_aggregate(x, src_f, dst_f, zf, zc, ones):
  mesh = plsc.VectorSubcoreMesh(
      core_axis_name="c", subcore_axis_name="s", num_cores=NC, num_subcores=NS)

  @functools.partial(
      pl.kernel,
      out_type=(
          jax.ShapeDtypeStruct((NC, N_PAD, D), jnp.float32),
          jax.ShapeDtypeStruct((NC, N_PAD), jnp.float32),
      ),
      mesh=mesh,
      scratch_types=[
          pltpu.VMEM((EPT_P,), jnp.int32),
          pltpu.VMEM((NCHUNKS, CHUNK), jnp.int32),
          pltpu.VMEM((2, CHUNK, D), jnp.float32),
          pltpu.VMEM((CHUNK,), jnp.float32),
          pltpu.VMEM_SHARED((N_PAD, D), jnp.float32),
          pltpu.VMEM_SHARED((N_PAD,), jnp.float32),
          pltpu.SemaphoreType.DMA,
          pltpu.SemaphoreType.DMA,
          pltpu.SemaphoreType.DMA,
      ],
  )
  def agg(x_hbm, src_hbm, dst_hbm, zf_hbm, zc_hbm, ones_hbm, of_hbm, oc_hbm,
          src_v, dst_v, rows_v, ones_v, acc_f, acc_c, sem0, sem1, semi):
    c = lax.axis_index("c")
    s = lax.axis_index("s")
    wid = c * NS + s
    row0 = s * ROWS_PT
    ebase = wid * EPT_P

    def fire(j, carry):
      pltpu.async_copy(dst_hbm.at[pl.ds(ebase + j * CHUNK, CHUNK)],
                       dst_v.at[j], semi)
      return carry
    lax.fori_loop(0, NCHUNKS, fire, 0)
    pltpu.sync_copy(src_hbm.at[pl.ds(ebase, EPT_P)], src_v)
    pltpu.sync_copy(ones_hbm, ones_v)

    def zero_rows(i, carry):
      r = row0 + i * 128
      pltpu.sync_copy(zf_hbm.at[pl.ds(r, 128)], acc_f.at[pl.ds(r, 128)])
      return carry
    lax.fori_loop(0, ROWS_PT // 128, zero_rows, 0)
    pltpu.sync_copy(zc_hbm.at[pl.ds(row0, ROWS_PT)],
                    acc_c.at[pl.ds(row0, ROWS_PT)])

    def drain(j, carry):
      pltpu.make_async_copy(dst_hbm.at[pl.ds(ebase + j * CHUNK, CHUNK)],
                            dst_v.at[j], semi).wait()
      return carry
    lax.fori_loop(0, NCHUNKS, drain, 0)

    pltpu.async_copy(x_hbm.at[src_v.at[pl.ds(0, CHUNK)]], rows_v.at[0], sem0)

    plsc.subcore_barrier()

    def step(j, carry):
      jm = lax.rem(j, 2)
      pltpu.make_async_copy(
          x_hbm.at[src_v.at[pl.ds(j * CHUNK, CHUNK)]], rows_v.at[jm],
          sem0).wait()

      @pl.when(j > 0)
      def _():
        jp = j - 1
        jpm = lax.rem(jp, 2)
        pltpu.make_async_copy(rows_v.at[jpm], acc_f.at[dst_v.at[jp]],
                              sem1).wait()
        pltpu.make_async_copy(ones_v, acc_c.at[dst_v.at[jp]], sem1).wait()

      pltpu.async_copy(rows_v.at[jm], acc_f.at[dst_v.at[j]], sem1, add=True)
      pltpu.async_copy(ones_v, acc_c.at[dst_v.at[j]], sem1, add=True)

      @pl.when(j + 1 < NCHUNKS)
      def _():
        jn = j + 1
        pltpu.async_copy(x_hbm.at[src_v.at[pl.ds(jn * CHUNK, CHUNK)]],
                         rows_v.at[lax.rem(jn, 2)], sem0)
      return carry
    lax.fori_loop(0, NCHUNKS, step, 0)

    jl = NCHUNKS - 1
    pltpu.make_async_copy(rows_v.at[lax.rem(jl, 2)], acc_f.at[dst_v.at[jl]],
                          sem1).wait()
    pltpu.make_async_copy(ones_v, acc_c.at[dst_v.at[jl]], sem1).wait()

    plsc.subcore_barrier()
    def write_rows(i, carry):
      r = row0 + i * 128
      pltpu.sync_copy(acc_f.at[pl.ds(r, 128)], of_hbm.at[c, pl.ds(r, 128)])
      return carry
    lax.fori_loop(0, ROWS_PT // 128, write_rows, 0)
    pltpu.sync_copy(acc_c.at[pl.ds(row0, ROWS_PT)],
                    oc_hbm.at[c, pl.ds(row0, ROWS_PT)])

  return agg(x, src_f, dst_f, zf, zc, ones)


def _tc_finish(x, pf, pc, W, b2):
  blk = 1000

  def body(pf_ref, pc_ref, x_ref, w_ref, b_ref, o_ref):
    ssum = pf_ref[0] + pf_ref[1]
    cnt = pc_ref[:, 0:1] + pc_ref[:, 1:2]
    mean = ssum / jnp.maximum(cnt, 1.0)
    h = jnp.where(cnt > 0.0, mean, x_ref[...])
    y = lax.dot_general(h, w_ref[...], (((1,), (1,)), ((), ())),
                        preferred_element_type=jnp.float32)
    o_ref[...] = jnp.maximum(y + b_ref[...], 0.0)

  return pl.pallas_call(
      body,
      grid=(N_NODES // blk,),
      in_specs=[
          pl.BlockSpec((2, blk, D), lambda i: (0, i, 0)),
          pl.BlockSpec((blk, 2), lambda i: (i, 0)),
          pl.BlockSpec((blk, D), lambda i: (i, 0)),
          pl.BlockSpec((D, D), lambda i: (0, 0)),
          pl.BlockSpec((1, D), lambda i: (0, 0)),
      ],
      out_specs=pl.BlockSpec((blk, D), lambda i: (i, 0)),
      out_shape=jax.ShapeDtypeStruct((N_NODES, D), jnp.float32),
  )(pf, pc, x, W, b2)


def kernel(x, edge_index, W, b):
  ei = edge_index.astype(jnp.int32)
  pad = ((0, 0), (0, EPT_P - EPT))
  src_f = jnp.pad(ei[0].reshape(NW, EPT), pad,
                  constant_values=0).reshape(NW * EPT_P)
  dst_f = jnp.pad(ei[1].reshape(NW, EPT), pad,
                  constant_values=N_NODES).reshape(NW * EPT_P)
  zf = jnp.zeros((N_PAD, D), jnp.float32)
  zc = jnp.zeros((N_PAD,), jnp.float32)
  ones = jnp.ones((CHUNK,), jnp.float32)
  pf, pc = _sc_aggregate(x, src_f, dst_f, zf, zc, ones)
  return _tc_finish(x, pf, pc.T, W, b.reshape(1, D))
